# baseline (speedup 1.0000x reference)
import functools

import jax
import jax.numpy as jnp
from jax import lax
from jax.experimental import pallas as pl
from jax.experimental.pallas import tpu as pltpu
from jax.experimental.pallas import tpu_sc as plsc

NUM_DEVICES = 8
NUM_EXPERTS = 16
TOP_K = 2
NC, NS, L = 2, 16, 16
NW = NC * NS
T = 4096
D_MODEL = 1024
S = T * TOP_K
SLOTS_PER_W = S // NW
ZROWS = 16
NZ = (NUM_DEVICES * SLOTS_PER_W) // ZROWS
NCHUNK = SLOTS_PER_W // L
TROWS = L // TOP_K
NBUF = 13


def _sc_dispatch(in_hbm, idx_hbm, map_hbm, zsrc_hbm, out_hbm,
                 zbuf, idx_v, map_v, rowid_v,
                 tb0, tb1, tb2, tb3, tb4, tb5, tb6, tb7, tb8, tb9, tb10, tb11, tb12,
                 zsem, tsems, ssems):
    wid = lax.axis_index("s") * NC + lax.axis_index("c")
    base = pl.multiple_of(wid * SLOTS_PER_W, SLOTS_PER_W)
    tbase = pl.multiple_of(base // TOP_K, SLOTS_PER_W // TOP_K)
    tbufs = [tb0, tb1, tb2, tb3, tb4, tb5, tb6, tb7, tb8, tb9, tb10, tb11, tb12]

    zstage = pltpu.make_async_copy(zsrc_hbm, zbuf, zsem)
    zstage.start()
    pltpu.sync_copy(idx_hbm.at[pl.ds(base, SLOTS_PER_W)], idx_v)
    pltpu.sync_copy(map_hbm, map_v)

    lane = jnp.arange(L, dtype=jnp.int32)
    for c in range(NCHUNK):
        i16 = idx_v[pl.ds(c * L, L)]
        dev16 = plsc.load_gather(map_v, [i16])
        slot16 = base + c * L + lane
        row16 = dev16 * S + slot16
        plsc.store_scatter(
            rowid_v,
            [2 * c + (lane & 1), lax.shift_right_logical(lane, 1)],
            row16,
        )
    zstage.wait()

    zcps = []
    for d in range(NUM_DEVICES):
        plane_base = d * S + base
        for z in range(SLOTS_PER_W // ZROWS):
            cp = pltpu.make_async_copy(
                zbuf,
                out_hbm.at[pl.ds(plane_base + z * ZROWS, ZROWS), :],
                zsem,
            )
            cp.start()
            zcps.append(cp)

    pend_t = {}
    for c in range(min(NBUF, NCHUNK)):
        t = pltpu.make_async_copy(
            in_hbm.at[pl.ds(tbase + c * TROWS, TROWS), :],
            tbufs[c % NBUF], tsems.at[c % NBUF])
        t.start()
        pend_t[c] = t

    for cp in zcps:
        cp.wait()

    for c in range(NCHUNK):
        b = c % NBUF
        pend_t[c].wait()
        s_ev = pltpu.make_async_copy(
            tbufs[b], out_hbm.at[rowid_v.at[2 * c]], ssems.at[b])
        s_od = pltpu.make_async_copy(
            tbufs[b], out_hbm.at[rowid_v.at[2 * c + 1]], ssems.at[b])
        s_ev.start()
        s_od.start()
        if c + NBUF < NCHUNK:
            s_ev.wait()
            s_od.wait()
            t = pltpu.make_async_copy(
                in_hbm.at[pl.ds(tbase + (c + NBUF) * TROWS, TROWS), :],
                tbufs[b], tsems.at[b])
            t.start()
            pend_t[c + NBUF] = t
        else:
            pend_t[c] = (s_ev, s_od)
    for c in range(NCHUNK - NBUF, NCHUNK):
        if c >= 0:
            s_ev, s_od = pend_t[c]
            s_ev.wait()
            s_od.wait()


def kernel(input_tensor, expert_indices, expert_mapping):
    idx_flat = expert_indices.reshape(-1)
    zsrc = jnp.zeros((ZROWS, D_MODEL), jnp.float32)
    mesh = plsc.VectorSubcoreMesh(core_axis_name="c", subcore_axis_name="s")
    k = functools.partial(
        pl.kernel,
        out_type=jax.ShapeDtypeStruct((NUM_DEVICES * S, D_MODEL), jnp.float32),
        mesh=mesh,
        compiler_params=pltpu.CompilerParams(needs_layout_passes=False),
        scratch_types=[
            pltpu.VMEM((ZROWS, D_MODEL), jnp.float32),
            pltpu.VMEM((SLOTS_PER_W,), jnp.int32),
            pltpu.VMEM((NUM_EXPERTS,), jnp.int32),
            pltpu.VMEM((2 * NCHUNK, TROWS), jnp.int32),
        ] + [pltpu.VMEM((TROWS, D_MODEL), jnp.float32)] * NBUF + [
            pltpu.SemaphoreType.DMA,
            pltpu.SemaphoreType.DMA((NBUF,)),
            pltpu.SemaphoreType.DMA((NBUF,)),
        ],
    )(_sc_dispatch)
    out = k(input_tensor, idx_flat, expert_mapping, zsrc)
    return out.reshape(NUM_DEVICES, S, D_MODEL)


# speedup vs baseline: 1.0104x; 1.0104x over previous
import functools

import jax
import jax.numpy as jnp
from jax import lax
from jax.experimental import pallas as pl
from jax.experimental.pallas import tpu as pltpu
from jax.experimental.pallas import tpu_sc as plsc

NUM_DEVICES = 8
NUM_EXPERTS = 16
TOP_K = 2
NC, NS, L = 2, 16, 16
NW = NC * NS
T = 4096
D_MODEL = 1024
S = T * TOP_K
SLOTS_PER_W = S // NW
ZROWS = 32
NZ = (NUM_DEVICES * SLOTS_PER_W) // ZROWS
NCHUNK = SLOTS_PER_W // L
TROWS = L // TOP_K
NBUF = 11


def _sc_dispatch(in_hbm, idx_hbm, map_hbm, zsrc_hbm, out_hbm,
                 zbuf, idx_v, map_v, rowid_v,
                 tb0, tb1, tb2, tb3, tb4, tb5, tb6, tb7, tb8, tb9, tb10,
                 zsem, tsems, ssems):
    wid = lax.axis_index("s") * NC + lax.axis_index("c")
    base = pl.multiple_of(wid * SLOTS_PER_W, SLOTS_PER_W)
    tbase = pl.multiple_of(base // TOP_K, SLOTS_PER_W // TOP_K)
    tbufs = [tb0, tb1, tb2, tb3, tb4, tb5, tb6, tb7, tb8, tb9, tb10]

    zstage = pltpu.make_async_copy(zsrc_hbm, zbuf, zsem)
    zstage.start()
    pltpu.sync_copy(idx_hbm.at[pl.ds(base, SLOTS_PER_W)], idx_v)
    pltpu.sync_copy(map_hbm, map_v)

    lane = jnp.arange(L, dtype=jnp.int32)
    for c in range(NCHUNK):
        i16 = idx_v[pl.ds(c * L, L)]
        dev16 = plsc.load_gather(map_v, [i16])
        slot16 = base + c * L + lane
        row16 = dev16 * S + slot16
        plsc.store_scatter(
            rowid_v,
            [2 * c + (lane & 1), lax.shift_right_logical(lane, 1)],
            row16,
        )
    zstage.wait()

    zcps = []
    for d in range(NUM_DEVICES):
        plane_base = d * S + base
        for z in range(SLOTS_PER_W // ZROWS):
            cp = pltpu.make_async_copy(
                zbuf,
                out_hbm.at[pl.ds(plane_base + z * ZROWS, ZROWS), :],
                zsem,
            )
            cp.start()
            zcps.append(cp)

    pend_t = {}
    for c in range(min(NBUF, NCHUNK)):
        t = pltpu.make_async_copy(
            in_hbm.at[pl.ds(tbase + c * TROWS, TROWS), :],
            tbufs[c % NBUF], tsems.at[c % NBUF])
        t.start()
        pend_t[c] = t

    for cp in zcps:
        cp.wait()

    for c in range(NCHUNK):
        b = c % NBUF
        pend_t[c].wait()
        s_ev = pltpu.make_async_copy(
            tbufs[b], out_hbm.at[rowid_v.at[2 * c]], ssems.at[b])
        s_od = pltpu.make_async_copy(
            tbufs[b], out_hbm.at[rowid_v.at[2 * c + 1]], ssems.at[b])
        s_ev.start()
        s_od.start()
        if c + NBUF < NCHUNK:
            s_ev.wait()
            s_od.wait()
            t = pltpu.make_async_copy(
                in_hbm.at[pl.ds(tbase + (c + NBUF) * TROWS, TROWS), :],
                tbufs[b], tsems.at[b])
            t.start()
            pend_t[c + NBUF] = t
        else:
            pend_t[c] = (s_ev, s_od)
    for c in range(NCHUNK - NBUF, NCHUNK):
        if c >= 0:
            s_ev, s_od = pend_t[c]
            s_ev.wait()
            s_od.wait()


def kernel(input_tensor, expert_indices, expert_mapping):
    idx_flat = expert_indices.reshape(-1)
    zsrc = jnp.zeros((ZROWS, D_MODEL), jnp.float32)
    mesh = plsc.VectorSubcoreMesh(core_axis_name="c", subcore_axis_name="s")
    k = functools.partial(
        pl.kernel,
        out_type=jax.ShapeDtypeStruct((NUM_DEVICES * S, D_MODEL), jnp.float32),
        mesh=mesh,
        compiler_params=pltpu.CompilerParams(needs_layout_passes=False),
        scratch_types=[
            pltpu.VMEM((ZROWS, D_MODEL), jnp.float32),
            pltpu.VMEM((SLOTS_PER_W,), jnp.int32),
            pltpu.VMEM((NUM_EXPERTS,), jnp.int32),
            pltpu.VMEM((2 * NCHUNK, TROWS), jnp.int32),
        ] + [pltpu.VMEM((TROWS, D_MODEL), jnp.float32)] * NBUF + [
            pltpu.SemaphoreType.DMA,
            pltpu.SemaphoreType.DMA((NBUF,)),
            pltpu.SemaphoreType.DMA((NBUF,)),
        ],
    )(_sc_dispatch)
    out = k(input_tensor, idx_flat, expert_mapping, zsrc)
    return out.reshape(NUM_DEVICES, S, D_MODEL)
